# baseline (speedup 1.0000x reference)
import functools

import jax
import jax.numpy as jnp
from jax.experimental import pallas as pl
from jax.experimental.pallas import tpu as pltpu

N_NODES = 10000
N_EDGES = 320000
NODE_DIM = 128
ER_DIM = 16
EA_DIM = 16
EC_DIM = ER_DIM + EA_DIM
MSG_DIM = NODE_DIM + EC_DIM

BE = 2000
BN = 2000


def _edge_mlp_body(srcf, dstf, src2f, er, ea, invsrc,
                   W1, b1, W2, b2, W3, b3, W4, b4, W5, b5, W6, b6,
                   edge_out, submsg):
    x = jnp.concatenate([srcf[...], dstf[...], er[...], ea[...]], axis=1)
    h = jax.nn.relu(jnp.dot(x, W1[...], preferred_element_type=jnp.float32) + b1[...])
    h = jax.nn.relu(jnp.dot(h, W2[...], preferred_element_type=jnp.float32) + b2[...])
    h = jax.nn.relu(jnp.dot(h, W3[...], preferred_element_type=jnp.float32) + b3[...])
    h = jax.nn.relu(jnp.dot(h, W4[...], preferred_element_type=jnp.float32) + b4[...])
    h = jax.nn.relu(jnp.dot(h, W5[...], preferred_element_type=jnp.float32) + b5[...])
    edge_out[...] = jnp.dot(h, W6[...], preferred_element_type=jnp.float32) + b6[...]
    inv = invsrc[...]
    submsg[...] = jnp.concatenate([src2f[...], er[...], ea[...]], axis=1) * inv


def _node_body(parts, nf, inv, Wn, bn, out):
    s = jnp.sum(parts[...], axis=0)
    invv = inv[...]
    rs = jnp.sqrt(invv)
    m128 = s[:, :NODE_DIM] * rs + nf[...] * invv
    m32 = s[:, NODE_DIM:] * rs
    msg = jnp.concatenate([m128, m32], axis=1)
    z = jnp.dot(msg, Wn[...], preferred_element_type=jnp.float32) + bn[...]
    out[...] = z * jax.nn.sigmoid(z)


def _full(shape):
    return pl.BlockSpec(shape, lambda i: tuple(0 for _ in shape))


def edge_mlp(srcf, dstf, src2f, er, ea, invsrc,
             W1, b1, W2, b2, W3, b3, W4, b4, W5, b5, W6, b6):
    nblk = N_EDGES // BE
    ws = [W1, b1, W2, b2, W3, b3, W4, b4, W5, b5, W6, b6]
    in_specs = [
        pl.BlockSpec((BE, NODE_DIM), lambda i: (i, 0)),
        pl.BlockSpec((BE, NODE_DIM), lambda i: (i, 0)),
        pl.BlockSpec((BE, NODE_DIM), lambda i: (i, 0)),
        pl.BlockSpec((BE, ER_DIM), lambda i: (i, 0)),
        pl.BlockSpec((BE, EA_DIM), lambda i: (i, 0)),
        pl.BlockSpec((BE, 1), lambda i: (i, 0)),
    ] + [_full(w.shape) for w in ws]
    out_specs = [
        pl.BlockSpec((BE, EC_DIM), lambda i: (i, 0)),
        pl.BlockSpec((BE, MSG_DIM), lambda i: (i, 0)),
    ]
    return pl.pallas_call(
        _edge_mlp_body,
        grid=(nblk,),
        in_specs=in_specs,
        out_specs=out_specs,
        out_shape=[
            jax.ShapeDtypeStruct((N_EDGES, EC_DIM), jnp.float32),
            jax.ShapeDtypeStruct((N_EDGES, MSG_DIM), jnp.float32),
        ],
    )(srcf, dstf, src2f, er, ea, invsrc, *ws)


def node_update(parts, nf, inv, Wn, bn):
    nblk = N_NODES // BN
    P = parts.shape[0]
    return pl.pallas_call(
        _node_body,
        grid=(nblk,),
        in_specs=[
            pl.BlockSpec((P, BN, MSG_DIM), lambda i: (0, i, 0)),
            pl.BlockSpec((BN, NODE_DIM), lambda i: (i, 0)),
            pl.BlockSpec((BN, 1), lambda i: (i, 0)),
            _full(Wn.shape),
            _full(bn.shape),
        ],
        out_specs=pl.BlockSpec((BN, NODE_DIM), lambda i: (i, 0)),
        out_shape=jax.ShapeDtypeStruct((N_NODES, NODE_DIM), jnp.float32),
    )(parts, nf, inv, Wn, bn)


@jax.jit
def kernel(node_features, edge_radial, edge_angular, edge_index,
           Wn, bn, W1, b1, W2, b2, W3, b3, W4, b4, W5, b5, W6, b6):
    src = edge_index[0].astype(jnp.int32)
    dst = edge_index[1].astype(jnp.int32)

    in_deg = jnp.zeros((N_NODES,), jnp.float32).at[dst].add(1.0)
    inv = 1.0 / in_deg

    srcf = node_features[src]
    dstf = node_features[dst]
    src2f = node_features[src[src]]
    invsrc = inv[src][:, None]

    edge_out, submsg = edge_mlp(
        srcf, dstf, src2f, edge_radial, edge_angular, invsrc,
        W1, b1, W2, b2, W3, b3, W4, b4, W5, b5, W6, b6)

    parts = jnp.zeros((1, N_NODES, MSG_DIM), jnp.float32).at[0, dst].add(submsg)

    node_out = node_update(parts, node_features, inv[:, None], Wn, bn)
    return (node_out, edge_out)
